# Initial kernel scaffold; baseline (speedup 1.0000x reference)
import jax
import jax.numpy as jnp
from jax.experimental import pallas as pl


def kernel(x, edge_index, W1, b1, W2, b2):
    raise NotImplementedError("write your pallas kernel here")


# speedup vs baseline: 4.0836x; 4.0836x over previous
import functools

import jax
import jax.numpy as jnp
from jax import lax
from jax.experimental import pallas as pl
from jax.experimental.pallas import tpu as pltpu
from jax.experimental.pallas import tpu_sc as plsc

N_NODES = 10000
N_EDGES = 320000
D = 128

NC, NS = 2, 16
HALF = N_NODES // NC
ACC_R = HALF + 8

DEG_W = 16
DEG_CH = 80
DEG_EPT = N_EDGES // NS
DEG_NCH = DEG_EPT // DEG_CH
DRPT = 624
DTAIL = N_NODES - NS * DRPT

EPT = N_EDGES // NS
CH = 80
EPAD = EPT + CH
ZR = 104
ARPT = 312
ATAIL = HALF - NS * ARPT

_sc_mesh = plsc.VectorSubcoreMesh(core_axis_name="c", subcore_axis_name="s")


HR = 128
HPT = HR // NS


@functools.partial(
    pl.kernel,
    out_type=jax.ShapeDtypeStruct((NC, HR, 128), jnp.float32),
    mesh=_sc_mesh,
    scratch_types=[
        pltpu.VMEM((DEG_EPT,), jnp.int32),
        pltpu.VMEM((HR, 128), jnp.float32),
        pltpu.VMEM((HPT, 128), jnp.float32),
        pltpu.VMEM((HPT, 128), jnp.float32),
        pltpu.VMEM_SHARED((NS, HR, 128), jnp.float32),
    ],
    compiler_params=pltpu.CompilerParams(needs_layout_passes=False),
)
def _deg_kernel(edges_hbm, out_hbm, idx_v, hist_v, acc_v, buf_v, stage_sh):
    cid = lax.axis_index("c")
    sid = lax.axis_index("s")

    one16 = jnp.ones((16,), jnp.float32)
    z16 = jnp.zeros((16,), jnp.float32)

    def fillz(i, _):
        hist_v[i // 8, pl.ds((i % 8) * 16, 16)] = z16
        return 0

    lax.fori_loop(0, HR * 8, fillz, 0)

    pltpu.sync_copy(
        edges_hbm.at[pl.ds(cid * N_EDGES + sid * DEG_EPT, DEG_EPT)], idx_v)

    def scan(i, _):
        idx = idx_v[pl.ds(i * 16, 16)]
        plsc.addupdate_scatter(hist_v, [idx >> 7, idx & 127], one16)
        return 0

    lax.fori_loop(0, DEG_EPT // 16, scan, 0)

    pltpu.sync_copy(hist_v, stage_sh.at[sid])
    plsc.subcore_barrier()

    def fillza(i, _):
        acc_v[i // 8, pl.ds((i % 8) * 16, 16)] = z16
        return 0

    lax.fori_loop(0, HPT * 8, fillza, 0)

    for s in range(NS):
        pltpu.sync_copy(stage_sh.at[s, pl.ds(sid * HPT, HPT)], buf_v)

        def addp(i, _):
            r, c = i // 8, (i % 8) * 16
            acc_v[r, pl.ds(c, 16)] = acc_v[r, pl.ds(c, 16)] + buf_v[r, pl.ds(c, 16)]
            return 0

        lax.fori_loop(0, HPT * 8, addp, 0)

    pltpu.sync_copy(acc_v, out_hbm.at[cid, pl.ds(sid * HPT, HPT)])


@functools.partial(
    pl.kernel,
    out_type=(
        jax.ShapeDtypeStruct((NC * NS * EPAD,), jnp.int32),
        jax.ShapeDtypeStruct((NC * NS * EPAD,), jnp.int32),
        jax.ShapeDtypeStruct((NC * NS * 16,), jnp.int32),
    ),
    mesh=_sc_mesh,
    scratch_types=[
        pltpu.VMEM((EPT,), jnp.int32),
        pltpu.VMEM((EPT,), jnp.int32),
        pltpu.VMEM((EPAD,), jnp.int32),
        pltpu.VMEM((EPAD,), jnp.int32),
        pltpu.VMEM((16,), jnp.int32),
    ],
    compiler_params=pltpu.CompilerParams(needs_layout_passes=False),
)
def _part_kernel(edges_hbm, srcc_hbm, dstc_hbm, cnt_hbm,
                 src_v, dst_v, srcc_v, dstc_v, cnt_v):
    cid = lax.axis_index("c")
    sid = lax.axis_index("s")

    pltpu.sync_copy(edges_hbm.at[pl.ds(sid * EPT, EPT)], src_v)
    pltpu.sync_copy(edges_hbm.at[pl.ds(N_EDGES + sid * EPT, EPT)], dst_v)

    lo = cid * HALF
    lo_vec = jnp.full((16,), 1, jnp.int32) * lo

    def scan(i, off):
        s16 = src_v[pl.ds(i * 16, 16)]
        d16 = dst_v[pl.ds(i * 16, 16)]
        dl = d16 - lo_vec
        m = (dl >= 0) & (dl < HALF)
        mi = m.astype(jnp.int32)
        pos = off + jnp.cumsum(mi) - 1
        plsc.store_scatter(srcc_v, [pos], s16, mask=m)
        plsc.store_scatter(dstc_v, [pos], dl, mask=m)
        return off + jnp.sum(mi)

    cnt = lax.fori_loop(0, EPT // 16, scan, jnp.int32(0))

    z16 = jnp.zeros((16,), jnp.int32)
    t16 = jnp.full((16,), 1, jnp.int32) * HALF
    for k in range(CH // 16):
        srcc_v[pl.ds(cnt + 16 * k, 16)] = z16
        dstc_v[pl.ds(cnt + 16 * k, 16)] = t16

    wbase = (cid * NS + sid) * EPAD
    pltpu.sync_copy(srcc_v, srcc_hbm.at[pl.ds(wbase, EPAD)])
    pltpu.sync_copy(dstc_v, dstc_hbm.at[pl.ds(wbase, EPAD)])
    cnt_v[...] = jnp.broadcast_to(cnt, (16,))
    pltpu.sync_copy(cnt_v, cnt_hbm.at[pl.ds((cid * NS + sid) * 16, 16)])


@functools.partial(
    pl.kernel,
    out_type=jax.ShapeDtypeStruct((N_NODES, D), jnp.float32),
    mesh=_sc_mesh,
    scratch_types=[
        pltpu.VMEM((CH,), jnp.int32),
        pltpu.VMEM((CH,), jnp.int32),
        pltpu.VMEM((CH, D), jnp.float32),
        pltpu.VMEM((ZR, D), jnp.float32),
        pltpu.VMEM((16,), jnp.int32),
        pltpu.VMEM_SHARED((ACC_R, D), jnp.float32),
        pltpu.SemaphoreType.DMA,
    ],
    compiler_params=pltpu.CompilerParams(needs_layout_passes=False),
)
def _agg_kernel(h_hbm, srcc_hbm, dstc_hbm, cnt_hbm, out_hbm,
                src_v, dst_v, rows_v, zz_v, cnt_v, acc_sh, sem):
    cid = lax.axis_index("c")
    sid = lax.axis_index("s")

    z16 = jnp.zeros((16,), jnp.float32)

    def fillz(t, _):
        zz_v[t // (D // 16), pl.ds((t % (D // 16)) * 16, 16)] = z16
        return 0

    lax.fori_loop(0, ZR * (D // 16), fillz, 0)

    for k in range(ARPT // ZR):
        pltpu.sync_copy(zz_v, acc_sh.at[pl.ds(sid * ARPT + k * ZR, ZR)])

    @pl.when(sid == NS - 1)
    def _():
        pltpu.sync_copy(zz_v.at[pl.ds(0, ATAIL)],
                        acc_sh.at[pl.ds(HALF - ATAIL, ATAIL)])

    plsc.subcore_barrier()

    wbase = (cid * NS + sid) * EPAD
    pltpu.sync_copy(cnt_hbm.at[pl.ds((cid * NS + sid) * 16, 16)], cnt_v)
    cnt = jnp.max(cnt_v[...])
    nch = (cnt + CH - 1) // CH

    def chunk(j, _):
        pltpu.sync_copy(srcc_hbm.at[pl.ds(wbase + j * CH, CH)], src_v)
        pltpu.sync_copy(dstc_hbm.at[pl.ds(wbase + j * CH, CH)], dst_v)
        pltpu.async_copy(h_hbm.at[src_v], rows_v, sem).wait()
        pltpu.sync_copy(rows_v, acc_sh.at[dst_v], add=True)
        return 0

    lax.fori_loop(0, nch, chunk, 0)
    plsc.subcore_barrier()

    pltpu.sync_copy(acc_sh.at[pl.ds(sid * ARPT, ARPT)],
                    out_hbm.at[pl.ds(cid * HALF + sid * ARPT, ARPT)])

    @pl.when(sid == NS - 1)
    def _():
        pltpu.sync_copy(acc_sh.at[pl.ds(HALF - ATAIL, ATAIL)],
                        out_hbm.at[pl.ds(cid * HALF + HALF - ATAIL, ATAIL)])


_BLK = 1000


def _norm(deg_blk):
    return lax.rsqrt(jnp.maximum(deg_blk[...], 1.0))


def _pre1_body(x_ref, dgo_ref, o_ref):
    o_ref[...] = x_ref[...] * _norm(dgo_ref)


def _post1_body(p_ref, dgi_ref, dgo_ref, w_ref, b_ref, o_ref):
    a = p_ref[...] * _norm(dgi_ref)
    z = jnp.dot(a, w_ref[...], preferred_element_type=jnp.float32) + b_ref[...]
    o_ref[...] = jnp.maximum(z, 0.0) * _norm(dgo_ref)


def _post2_body(p_ref, dgi_ref, w_ref, b_ref, o_ref):
    a = p_ref[...] * _norm(dgi_ref)
    o_ref[...] = jnp.dot(a, w_ref[...], preferred_element_type=jnp.float32) + b_ref[...]


def _row_blocked(width):
    return pl.BlockSpec((_BLK, width), lambda i: (i, 0))


_SPEC_W = pl.BlockSpec((D, D), lambda i: (0, 0))
_SPEC_B = pl.BlockSpec((1, D), lambda i: (0, 0))
_OUT_SHAPE = jax.ShapeDtypeStruct((N_NODES, D), jnp.float32)
_GRID = (N_NODES // _BLK,)


def kernel(x, edge_index, W1, b1, W2, b2):
    eflat = edge_index.reshape(2 * N_EDGES)
    degp = _deg_kernel(eflat).reshape(NC, HR * 128)[:, :N_NODES]
    dgo = degp[0][:, None]
    dgi = degp[1][:, None]

    srcc, dstc, cnts = _part_kernel(eflat)

    h1 = pl.pallas_call(
        _pre1_body,
        grid=_GRID,
        in_specs=[_row_blocked(D), _row_blocked(1)],
        out_specs=_row_blocked(D),
        out_shape=_OUT_SHAPE,
    )(x, dgo)

    p1 = _agg_kernel(h1, srcc, dstc, cnts)

    h2 = pl.pallas_call(
        _post1_body,
        grid=_GRID,
        in_specs=[_row_blocked(D), _row_blocked(1), _row_blocked(1),
                  _SPEC_W, _SPEC_B],
        out_specs=_row_blocked(D),
        out_shape=_OUT_SHAPE,
    )(p1, dgi, dgo, W1, b1.reshape(1, D))

    p2 = _agg_kernel(h2, srcc, dstc, cnts)

    out = pl.pallas_call(
        _post2_body,
        grid=_GRID,
        in_specs=[_row_blocked(D), _row_blocked(1), _SPEC_W, _SPEC_B],
        out_specs=_row_blocked(D),
        out_shape=_OUT_SHAPE,
    )(p2, dgi, W2, b2.reshape(1, D))

    return out


# speedup vs baseline: 6.7588x; 1.6551x over previous
import functools

import jax
import jax.numpy as jnp
from jax import lax
from jax.experimental import pallas as pl
from jax.experimental.pallas import tpu as pltpu
from jax.experimental.pallas import tpu_sc as plsc

N_NODES = 10000
N_EDGES = 320000
D = 128

NC, NS = 2, 16
HALF = N_NODES // NC
ACC_R = HALF + 8

DEG_W = 16
DEG_CH = 80
DEG_EPT = N_EDGES // NS
DEG_NCH = DEG_EPT // DEG_CH
DRPT = 624
DTAIL = N_NODES - NS * DRPT

EPT = N_EDGES // NS
CH = 128
NCHMAX = 160
EPAD = NCHMAX * CH
ZR = 104
ARPT = 312
ATAIL = HALF - NS * ARPT

_sc_mesh = plsc.VectorSubcoreMesh(core_axis_name="c", subcore_axis_name="s")


HR = 128
HPT = HR // NS


@functools.partial(
    pl.kernel,
    out_type=jax.ShapeDtypeStruct((NC, HR, 128), jnp.float32),
    mesh=_sc_mesh,
    scratch_types=[
        pltpu.VMEM((DEG_EPT,), jnp.int32),
        pltpu.VMEM((HR, 128), jnp.float32),
        pltpu.VMEM((HPT, 128), jnp.float32),
        pltpu.VMEM((HPT, 128), jnp.float32),
        pltpu.VMEM_SHARED((NS, HR, 128), jnp.float32),
    ],
    compiler_params=pltpu.CompilerParams(needs_layout_passes=False),
)
def _deg_kernel(edges_hbm, out_hbm, idx_v, hist_v, acc_v, buf_v, stage_sh):
    cid = lax.axis_index("c")
    sid = lax.axis_index("s")

    one16 = jnp.ones((16,), jnp.float32)
    z16 = jnp.zeros((16,), jnp.float32)

    def fillz(i, _):
        hist_v[i // 8, pl.ds((i % 8) * 16, 16)] = z16
        return 0

    lax.fori_loop(0, HR * 8, fillz, 0)

    pltpu.sync_copy(
        edges_hbm.at[pl.ds(cid * N_EDGES + sid * DEG_EPT, DEG_EPT)], idx_v)

    def scan(i, _):
        idx = idx_v[pl.ds(i * 16, 16)]
        plsc.addupdate_scatter(hist_v, [idx >> 7, idx & 127], one16)
        return 0

    lax.fori_loop(0, DEG_EPT // 16, scan, 0)

    pltpu.sync_copy(hist_v, stage_sh.at[sid])
    plsc.subcore_barrier()

    def fillza(i, _):
        acc_v[i // 8, pl.ds((i % 8) * 16, 16)] = z16
        return 0

    lax.fori_loop(0, HPT * 8, fillza, 0)

    for s in range(NS):
        pltpu.sync_copy(stage_sh.at[s, pl.ds(sid * HPT, HPT)], buf_v)

        def addp(i, _):
            r, c = i // 8, (i % 8) * 16
            acc_v[r, pl.ds(c, 16)] = acc_v[r, pl.ds(c, 16)] + buf_v[r, pl.ds(c, 16)]
            return 0

        lax.fori_loop(0, HPT * 8, addp, 0)

    pltpu.sync_copy(acc_v, out_hbm.at[cid, pl.ds(sid * HPT, HPT)])


@functools.partial(
    pl.kernel,
    out_type=(
        jax.ShapeDtypeStruct((NC * NS, NCHMAX, CH), jnp.int32),
        jax.ShapeDtypeStruct((NC * NS, NCHMAX, CH), jnp.int32),
        jax.ShapeDtypeStruct((NC * NS * 16,), jnp.int32),
    ),
    mesh=_sc_mesh,
    scratch_types=[
        pltpu.VMEM((EPT,), jnp.int32),
        pltpu.VMEM((EPT,), jnp.int32),
        pltpu.VMEM((NCHMAX, CH), jnp.int32),
        pltpu.VMEM((NCHMAX, CH), jnp.int32),
        pltpu.VMEM((16,), jnp.int32),
    ],
    compiler_params=pltpu.CompilerParams(needs_layout_passes=False),
)
def _part_kernel(edges_hbm, srcc_hbm, dstc_hbm, cnt_hbm,
                 src_v, dst_v, srcc_v, dstc_v, cnt_v):
    cid = lax.axis_index("c")
    sid = lax.axis_index("s")

    pltpu.sync_copy(edges_hbm.at[pl.ds(sid * EPT, EPT)], src_v)
    pltpu.sync_copy(edges_hbm.at[pl.ds(N_EDGES + sid * EPT, EPT)], dst_v)

    lo = cid * HALF
    lo_vec = jnp.full((16,), 1, jnp.int32) * lo

    def scan(i, off):
        s16 = src_v[pl.ds(i * 16, 16)]
        d16 = dst_v[pl.ds(i * 16, 16)]
        dl = d16 - lo_vec
        m = (dl >= 0) & (dl < HALF)
        mi = m.astype(jnp.int32)
        pos = off + jnp.cumsum(mi) - 1
        plsc.store_scatter(srcc_v, [pos >> 7, pos & 127], s16, mask=m)
        plsc.store_scatter(dstc_v, [pos >> 7, pos & 127], dl, mask=m)
        return off + jnp.sum(mi)

    cnt = lax.fori_loop(0, EPT // 16, scan, jnp.int32(0))

    z16 = jnp.zeros((16,), jnp.int32)
    t16 = jnp.full((16,), 1, jnp.int32) * HALF
    lane = lax.iota(jnp.int32, 16)
    for k in range(CH // 16):
        pos = cnt + 16 * k + lane
        plsc.store_scatter(srcc_v, [pos >> 7, pos & 127], z16)
        plsc.store_scatter(dstc_v, [pos >> 7, pos & 127], t16)

    wid = cid * NS + sid
    pltpu.sync_copy(srcc_v, srcc_hbm.at[wid, pl.ds(0, NCHMAX)])
    pltpu.sync_copy(dstc_v, dstc_hbm.at[wid, pl.ds(0, NCHMAX)])
    cnt_v[...] = jnp.broadcast_to(cnt, (16,))
    pltpu.sync_copy(cnt_v, cnt_hbm.at[pl.ds((cid * NS + sid) * 16, 16)])


@functools.partial(
    pl.kernel,
    out_type=jax.ShapeDtypeStruct((N_NODES, D), jnp.float32),
    mesh=_sc_mesh,
    scratch_types=[
        pltpu.VMEM((NCHMAX, CH), jnp.int32),
        pltpu.VMEM((NCHMAX, CH), jnp.int32),
        pltpu.VMEM((2, CH, D), jnp.float32),
        pltpu.VMEM((ZR, D), jnp.float32),
        pltpu.VMEM((16,), jnp.int32),
        pltpu.VMEM_SHARED((ACC_R, D), jnp.float32),
        pltpu.SemaphoreType.DMA,
    ],
    compiler_params=pltpu.CompilerParams(needs_layout_passes=False),
)
def _agg_kernel(h_hbm, srcc_hbm, dstc_hbm, cnt_hbm, out_hbm,
                src_v, dst_v, rows_v, zz_v, cnt_v, acc_sh, sem):
    cid = lax.axis_index("c")
    sid = lax.axis_index("s")

    z16 = jnp.zeros((16,), jnp.float32)

    def fillz(t, _):
        zz_v[t // (D // 16), pl.ds((t % (D // 16)) * 16, 16)] = z16
        return 0

    lax.fori_loop(0, ZR * (D // 16), fillz, 0)

    for k in range(ARPT // ZR):
        pltpu.sync_copy(zz_v, acc_sh.at[pl.ds(sid * ARPT + k * ZR, ZR)])

    @pl.when(sid == NS - 1)
    def _():
        pltpu.sync_copy(zz_v.at[pl.ds(0, ATAIL)],
                        acc_sh.at[pl.ds(HALF - ATAIL, ATAIL)])

    plsc.subcore_barrier()

    wid = cid * NS + sid
    pltpu.sync_copy(cnt_hbm.at[pl.ds(wid * 16, 16)], cnt_v)
    cnt = jnp.max(cnt_v[...])
    nch = (cnt + CH - 1) // CH

    pltpu.sync_copy(srcc_hbm.at[wid, pl.ds(0, NCHMAX)], src_v)
    pltpu.sync_copy(dstc_hbm.at[wid, pl.ds(0, NCHMAX)], dst_v)

    @pl.when(nch > 0)
    def _():
        pltpu.async_copy(h_hbm.at[src_v.at[0, pl.ds(0, CH)]], rows_v.at[0, pl.ds(0, CH)], sem)

    def chunk(j, _):
        @pl.when(j + 1 < nch)
        def _():
            pltpu.async_copy(h_hbm.at[src_v.at[j + 1, pl.ds(0, CH)]], rows_v.at[(j + 1) & 1, pl.ds(0, CH)], sem)
        pltpu.make_async_copy(h_hbm.at[src_v.at[j, pl.ds(0, CH)]], rows_v.at[j & 1, pl.ds(0, CH)], sem).wait()
        pltpu.sync_copy(rows_v.at[j & 1, pl.ds(0, CH)], acc_sh.at[dst_v.at[j, pl.ds(0, CH)]], add=True)
        return 0

    lax.fori_loop(0, nch, chunk, 0)
    plsc.subcore_barrier()

    pltpu.sync_copy(acc_sh.at[pl.ds(sid * ARPT, ARPT)],
                    out_hbm.at[pl.ds(cid * HALF + sid * ARPT, ARPT)])

    @pl.when(sid == NS - 1)
    def _():
        pltpu.sync_copy(acc_sh.at[pl.ds(HALF - ATAIL, ATAIL)],
                        out_hbm.at[pl.ds(cid * HALF + HALF - ATAIL, ATAIL)])


_BLK = 1000


def _norm(deg_blk):
    return lax.rsqrt(jnp.maximum(deg_blk[...], 1.0))


def _pre1_body(x_ref, dgo_ref, o_ref):
    o_ref[...] = x_ref[...] * _norm(dgo_ref)


def _post1_body(p_ref, dgi_ref, dgo_ref, w_ref, b_ref, o_ref):
    a = p_ref[...] * _norm(dgi_ref)
    z = jnp.dot(a, w_ref[...], preferred_element_type=jnp.float32) + b_ref[...]
    o_ref[...] = jnp.maximum(z, 0.0) * _norm(dgo_ref)


def _post2_body(p_ref, dgi_ref, w_ref, b_ref, o_ref):
    a = p_ref[...] * _norm(dgi_ref)
    o_ref[...] = jnp.dot(a, w_ref[...], preferred_element_type=jnp.float32) + b_ref[...]


def _row_blocked(width):
    return pl.BlockSpec((_BLK, width), lambda i: (i, 0))


_SPEC_W = pl.BlockSpec((D, D), lambda i: (0, 0))
_SPEC_B = pl.BlockSpec((1, D), lambda i: (0, 0))
_OUT_SHAPE = jax.ShapeDtypeStruct((N_NODES, D), jnp.float32)
_GRID = (N_NODES // _BLK,)


def kernel(x, edge_index, W1, b1, W2, b2):
    eflat = edge_index.reshape(2 * N_EDGES)
    degp = _deg_kernel(eflat).reshape(NC, HR * 128)[:, :N_NODES]
    dgo = degp[0][:, None]
    dgi = degp[1][:, None]

    srcc, dstc, cnts = _part_kernel(eflat)

    h1 = pl.pallas_call(
        _pre1_body,
        grid=_GRID,
        in_specs=[_row_blocked(D), _row_blocked(1)],
        out_specs=_row_blocked(D),
        out_shape=_OUT_SHAPE,
    )(x, dgo)

    p1 = _agg_kernel(h1, srcc, dstc, cnts)

    h2 = pl.pallas_call(
        _post1_body,
        grid=_GRID,
        in_specs=[_row_blocked(D), _row_blocked(1), _row_blocked(1),
                  _SPEC_W, _SPEC_B],
        out_specs=_row_blocked(D),
        out_shape=_OUT_SHAPE,
    )(p1, dgi, dgo, W1, b1.reshape(1, D))

    p2 = _agg_kernel(h2, srcc, dstc, cnts)

    out = pl.pallas_call(
        _post2_body,
        grid=_GRID,
        in_specs=[_row_blocked(D), _row_blocked(1), _SPEC_W, _SPEC_B],
        out_specs=_row_blocked(D),
        out_shape=_OUT_SHAPE,
    )(p2, dgi, W2, b2.reshape(1, D))

    return out


# speedup vs baseline: 7.0343x; 1.0408x over previous
import functools

import jax
import jax.numpy as jnp
from jax import lax
from jax.experimental import pallas as pl
from jax.experimental.pallas import tpu as pltpu
from jax.experimental.pallas import tpu_sc as plsc

N_NODES = 10000
N_EDGES = 320000
D = 128

NC, NS = 2, 16
HALF = N_NODES // NC
ACC_R = HALF + 8

DEG_W = 16
DEG_CH = 80
DEG_EPT = N_EDGES // NS
DEG_NCH = DEG_EPT // DEG_CH
DRPT = 624
DTAIL = N_NODES - NS * DRPT

EPT = N_EDGES // NS
CH = 128
NCHMAX = 158
EPAD = NCHMAX * CH
ZR = 8
ARPT = 312
ATAIL = HALF - NS * ARPT

_sc_mesh = plsc.VectorSubcoreMesh(core_axis_name="c", subcore_axis_name="s")


HR = 128
HPT = HR // NS


@functools.partial(
    pl.kernel,
    out_type=jax.ShapeDtypeStruct((NC, HR, 128), jnp.float32),
    mesh=_sc_mesh,
    scratch_types=[
        pltpu.VMEM((DEG_EPT,), jnp.int32),
        pltpu.VMEM((HR, 128), jnp.float32),
        pltpu.VMEM((HPT, 128), jnp.float32),
        pltpu.VMEM((HPT, 128), jnp.float32),
        pltpu.VMEM_SHARED((NS, HR, 128), jnp.float32),
    ],
    compiler_params=pltpu.CompilerParams(needs_layout_passes=False),
)
def _deg_kernel(edges_hbm, out_hbm, idx_v, hist_v, acc_v, buf_v, stage_sh):
    cid = lax.axis_index("c")
    sid = lax.axis_index("s")

    one16 = jnp.ones((16,), jnp.float32)
    z16 = jnp.zeros((16,), jnp.float32)

    def fillz(i, _):
        hist_v[i // 8, pl.ds((i % 8) * 16, 16)] = z16
        return 0

    lax.fori_loop(0, HR * 8, fillz, 0)

    pltpu.sync_copy(
        edges_hbm.at[pl.ds(cid * N_EDGES + sid * DEG_EPT, DEG_EPT)], idx_v)

    def scan(i, _):
        idx = idx_v[pl.ds(i * 16, 16)]
        plsc.addupdate_scatter(hist_v, [idx >> 7, idx & 127], one16)
        return 0

    lax.fori_loop(0, DEG_EPT // 16, scan, 0)

    pltpu.sync_copy(hist_v, stage_sh.at[sid])
    plsc.subcore_barrier()

    def fillza(i, _):
        acc_v[i // 8, pl.ds((i % 8) * 16, 16)] = z16
        return 0

    lax.fori_loop(0, HPT * 8, fillza, 0)

    for s in range(NS):
        pltpu.sync_copy(stage_sh.at[s, pl.ds(sid * HPT, HPT)], buf_v)

        def addp(i, _):
            r, c = i // 8, (i % 8) * 16
            acc_v[r, pl.ds(c, 16)] = acc_v[r, pl.ds(c, 16)] + buf_v[r, pl.ds(c, 16)]
            return 0

        lax.fori_loop(0, HPT * 8, addp, 0)

    pltpu.sync_copy(acc_v, out_hbm.at[cid, pl.ds(sid * HPT, HPT)])


@functools.partial(
    pl.kernel,
    out_type=(
        jax.ShapeDtypeStruct((NC * NS, NCHMAX, CH), jnp.int32),
        jax.ShapeDtypeStruct((NC * NS, NCHMAX, CH), jnp.int32),
        jax.ShapeDtypeStruct((NC * NS * 16,), jnp.int32),
    ),
    mesh=_sc_mesh,
    scratch_types=[
        pltpu.VMEM((EPT,), jnp.int32),
        pltpu.VMEM((EPT,), jnp.int32),
        pltpu.VMEM((NCHMAX, CH), jnp.int32),
        pltpu.VMEM((NCHMAX, CH), jnp.int32),
        pltpu.VMEM((16,), jnp.int32),
    ],
    compiler_params=pltpu.CompilerParams(needs_layout_passes=False),
)
def _part_kernel(edges_hbm, srcc_hbm, dstc_hbm, cnt_hbm,
                 src_v, dst_v, srcc_v, dstc_v, cnt_v):
    cid = lax.axis_index("c")
    sid = lax.axis_index("s")

    pltpu.sync_copy(edges_hbm.at[pl.ds(sid * EPT, EPT)], src_v)
    pltpu.sync_copy(edges_hbm.at[pl.ds(N_EDGES + sid * EPT, EPT)], dst_v)

    lo = cid * HALF
    lo_vec = jnp.full((16,), 1, jnp.int32) * lo

    def scan(i, off):
        s16 = src_v[pl.ds(i * 16, 16)]
        d16 = dst_v[pl.ds(i * 16, 16)]
        dl = d16 - lo_vec
        m = (dl >= 0) & (dl < HALF)
        mi = m.astype(jnp.int32)
        pos = off + jnp.cumsum(mi) - 1
        plsc.store_scatter(srcc_v, [pos >> 7, pos & 127], s16, mask=m)
        plsc.store_scatter(dstc_v, [pos >> 7, pos & 127], dl, mask=m)
        return off + jnp.sum(mi)

    cnt = lax.fori_loop(0, EPT // 16, scan, jnp.int32(0))

    z16 = jnp.zeros((16,), jnp.int32)
    t16 = jnp.full((16,), 1, jnp.int32) * HALF
    lane = lax.iota(jnp.int32, 16)
    for k in range(CH // 16):
        pos = cnt + 16 * k + lane
        plsc.store_scatter(srcc_v, [pos >> 7, pos & 127], z16)
        plsc.store_scatter(dstc_v, [pos >> 7, pos & 127], t16)

    wid = cid * NS + sid
    pltpu.sync_copy(srcc_v, srcc_hbm.at[wid, pl.ds(0, NCHMAX)])
    pltpu.sync_copy(dstc_v, dstc_hbm.at[wid, pl.ds(0, NCHMAX)])
    cnt_v[...] = jnp.broadcast_to(cnt, (16,))
    pltpu.sync_copy(cnt_v, cnt_hbm.at[pl.ds((cid * NS + sid) * 16, 16)])


@functools.partial(
    pl.kernel,
    out_type=jax.ShapeDtypeStruct((N_NODES, D), jnp.float32),
    mesh=_sc_mesh,
    scratch_types=[
        pltpu.VMEM((NCHMAX, CH), jnp.int32),
        pltpu.VMEM((NCHMAX, CH), jnp.int32),
        pltpu.VMEM((3, CH, D), jnp.float32),
        pltpu.VMEM((16,), jnp.int32),
        pltpu.VMEM_SHARED((ACC_R, D), jnp.float32),
        pltpu.SemaphoreType.DMA,
        pltpu.SemaphoreType.DMA,
    ],
    compiler_params=pltpu.CompilerParams(needs_layout_passes=False),
)
def _agg_kernel(h_hbm, srcc_hbm, dstc_hbm, cnt_hbm, out_hbm,
                src_v, dst_v, rows_v, cnt_v, acc_sh, sem_g, sem_s):
    cid = lax.axis_index("c")
    sid = lax.axis_index("s")

    z16 = jnp.zeros((16,), jnp.float32)

    def fillz(t, _):
        rows_v[0, t // (D // 16), pl.ds((t % (D // 16)) * 16, 16)] = z16
        return 0

    lax.fori_loop(0, CH * (D // 16), fillz, 0)

    pltpu.sync_copy(rows_v.at[0, pl.ds(0, CH)],
                    acc_sh.at[pl.ds(sid * ARPT, CH)])
    pltpu.sync_copy(rows_v.at[0, pl.ds(0, CH)],
                    acc_sh.at[pl.ds(sid * ARPT + CH, CH)])
    pltpu.sync_copy(rows_v.at[0, pl.ds(0, ARPT - 2 * CH)],
                    acc_sh.at[pl.ds(sid * ARPT + 2 * CH, ARPT - 2 * CH)])

    @pl.when(sid == NS - 1)
    def _():
        pltpu.sync_copy(rows_v.at[0, pl.ds(0, ATAIL)],
                        acc_sh.at[pl.ds(HALF - ATAIL, ATAIL)])

    plsc.subcore_barrier()

    wid = cid * NS + sid
    pltpu.sync_copy(cnt_hbm.at[pl.ds(wid * 16, 16)], cnt_v)
    cnt = jnp.max(cnt_v[...])
    nch = (cnt + CH - 1) // CH

    pltpu.sync_copy(srcc_hbm.at[wid, pl.ds(0, NCHMAX)], src_v)
    pltpu.sync_copy(dstc_hbm.at[wid, pl.ds(0, NCHMAX)], dst_v)

    def _gather(f):
        pltpu.async_copy(h_hbm.at[src_v.at[f, pl.ds(0, CH)]],
                         rows_v.at[f % 3, pl.ds(0, CH)], sem_g)

    def _wait_g(f):
        pltpu.make_async_copy(h_hbm.at[src_v.at[f, pl.ds(0, CH)]],
                              rows_v.at[f % 3, pl.ds(0, CH)], sem_g).wait()

    def _scatter(f):
        pltpu.async_copy(rows_v.at[f % 3, pl.ds(0, CH)],
                         acc_sh.at[dst_v.at[f, pl.ds(0, CH)]], sem_s, add=True)

    def _wait_s():
        pltpu.make_async_copy(rows_v.at[0, pl.ds(0, CH)],
                              acc_sh.at[dst_v.at[0, pl.ds(0, CH)]], sem_s).wait()

    for k in range(3):
        @pl.when(k < nch)
        def _():
            _gather(jnp.int32(k))

    def chunk(j, _):
        _wait_g(j)
        _scatter(j)

        @pl.when(j >= 1)
        def _():
            _wait_s()

            @pl.when(j + 2 < nch)
            def _():
                _gather(j + 2)
        return 0

    lax.fori_loop(0, nch, chunk, 0)

    @pl.when(nch >= 1)
    def _():
        _wait_s()
    plsc.subcore_barrier()

    pltpu.sync_copy(acc_sh.at[pl.ds(sid * ARPT, ARPT)],
                    out_hbm.at[pl.ds(cid * HALF + sid * ARPT, ARPT)])

    @pl.when(sid == NS - 1)
    def _():
        pltpu.sync_copy(acc_sh.at[pl.ds(HALF - ATAIL, ATAIL)],
                        out_hbm.at[pl.ds(cid * HALF + HALF - ATAIL, ATAIL)])


_BLK = 1000


def _norm(deg_blk):
    return lax.rsqrt(jnp.maximum(deg_blk[...], 1.0))


def _pre1_body(x_ref, dgo_ref, o_ref):
    o_ref[...] = x_ref[...] * _norm(dgo_ref)


def _post1_body(p_ref, dgi_ref, dgo_ref, w_ref, b_ref, o_ref):
    a = p_ref[...] * _norm(dgi_ref)
    z = jnp.dot(a, w_ref[...], preferred_element_type=jnp.float32) + b_ref[...]
    o_ref[...] = jnp.maximum(z, 0.0) * _norm(dgo_ref)


def _post2_body(p_ref, dgi_ref, w_ref, b_ref, o_ref):
    a = p_ref[...] * _norm(dgi_ref)
    o_ref[...] = jnp.dot(a, w_ref[...], preferred_element_type=jnp.float32) + b_ref[...]


def _row_blocked(width):
    return pl.BlockSpec((_BLK, width), lambda i: (i, 0))


_SPEC_W = pl.BlockSpec((D, D), lambda i: (0, 0))
_SPEC_B = pl.BlockSpec((1, D), lambda i: (0, 0))
_OUT_SHAPE = jax.ShapeDtypeStruct((N_NODES, D), jnp.float32)
_GRID = (N_NODES // _BLK,)


def kernel(x, edge_index, W1, b1, W2, b2):
    eflat = edge_index.reshape(2 * N_EDGES)
    degp = _deg_kernel(eflat).reshape(NC, HR * 128)[:, :N_NODES]
    dgo = degp[0][:, None]
    dgi = degp[1][:, None]

    srcc, dstc, cnts = _part_kernel(eflat)

    h1 = pl.pallas_call(
        _pre1_body,
        grid=_GRID,
        in_specs=[_row_blocked(D), _row_blocked(1)],
        out_specs=_row_blocked(D),
        out_shape=_OUT_SHAPE,
    )(x, dgo)

    p1 = _agg_kernel(h1, srcc, dstc, cnts)

    h2 = pl.pallas_call(
        _post1_body,
        grid=_GRID,
        in_specs=[_row_blocked(D), _row_blocked(1), _row_blocked(1),
                  _SPEC_W, _SPEC_B],
        out_specs=_row_blocked(D),
        out_shape=_OUT_SHAPE,
    )(p1, dgi, dgo, W1, b1.reshape(1, D))

    p2 = _agg_kernel(h2, srcc, dstc, cnts)

    out = pl.pallas_call(
        _post2_body,
        grid=_GRID,
        in_specs=[_row_blocked(D), _row_blocked(1), _SPEC_W, _SPEC_B],
        out_specs=_row_blocked(D),
        out_shape=_OUT_SHAPE,
    )(p2, dgi, W2, b2.reshape(1, D))

    return out


# speedup vs baseline: 7.0986x; 1.0091x over previous
import functools

import jax
import jax.numpy as jnp
from jax import lax
from jax.experimental import pallas as pl
from jax.experimental.pallas import tpu as pltpu
from jax.experimental.pallas import tpu_sc as plsc

N_NODES = 10000
N_EDGES = 320000
D = 128

NC, NS = 2, 16
HALF = N_NODES // NC
ACC_R = HALF + 8

DEG_W = 16
DEG_CH = 80
DEG_EPT = N_EDGES // NS
DEG_NCH = DEG_EPT // DEG_CH
DRPT = 624
DTAIL = N_NODES - NS * DRPT

EPT = N_EDGES // NS
CH = 128
NCHMAX = 158
EPAD = NCHMAX * CH
ZR = 8
ARPT = 312
ATAIL = HALF - NS * ARPT

_sc_mesh = plsc.VectorSubcoreMesh(core_axis_name="c", subcore_axis_name="s")


HR = 128
HPT = HR // NS


@functools.partial(
    pl.kernel,
    out_type=jax.ShapeDtypeStruct((NC, HR, 128), jnp.float32),
    mesh=_sc_mesh,
    scratch_types=[
        pltpu.VMEM((DEG_EPT,), jnp.int32),
        pltpu.VMEM((HR, 128), jnp.float32),
        pltpu.VMEM((HPT, 128), jnp.float32),
        pltpu.VMEM((HPT, 128), jnp.float32),
        pltpu.VMEM_SHARED((NS, HR, 128), jnp.float32),
    ],
    compiler_params=pltpu.CompilerParams(needs_layout_passes=False),
)
def _deg_kernel(edges_hbm, out_hbm, idx_v, hist_v, acc_v, buf_v, stage_sh):
    cid = lax.axis_index("c")
    sid = lax.axis_index("s")

    one16 = jnp.ones((16,), jnp.float32)
    z16 = jnp.zeros((16,), jnp.float32)

    def fillz(i, _):
        hist_v[i // 8, pl.ds((i % 8) * 16, 16)] = z16
        return 0

    lax.fori_loop(0, HR * 8, fillz, 0)

    pltpu.sync_copy(
        edges_hbm.at[pl.ds(cid * N_EDGES + sid * DEG_EPT, DEG_EPT)], idx_v)

    def scan(i, _):
        idx = idx_v[pl.ds(i * 16, 16)]
        plsc.addupdate_scatter(hist_v, [idx >> 7, idx & 127], one16)
        return 0

    lax.fori_loop(0, DEG_EPT // 16, scan, 0)

    pltpu.sync_copy(hist_v, stage_sh.at[sid])
    plsc.subcore_barrier()

    def fillza(i, _):
        acc_v[i // 8, pl.ds((i % 8) * 16, 16)] = z16
        return 0

    lax.fori_loop(0, HPT * 8, fillza, 0)

    for s in range(NS):
        pltpu.sync_copy(stage_sh.at[s, pl.ds(sid * HPT, HPT)], buf_v)

        def addp(i, _):
            r, c = i // 8, (i % 8) * 16
            acc_v[r, pl.ds(c, 16)] = acc_v[r, pl.ds(c, 16)] + buf_v[r, pl.ds(c, 16)]
            return 0

        lax.fori_loop(0, HPT * 8, addp, 0)

    pltpu.sync_copy(acc_v, out_hbm.at[cid, pl.ds(sid * HPT, HPT)])


@functools.partial(
    pl.kernel,
    out_type=(
        jax.ShapeDtypeStruct((NC * NS, NCHMAX, CH), jnp.int32),
        jax.ShapeDtypeStruct((NC * NS, NCHMAX, CH), jnp.int32),
        jax.ShapeDtypeStruct((NC * NS * 16,), jnp.int32),
    ),
    mesh=_sc_mesh,
    scratch_types=[
        pltpu.VMEM((EPT,), jnp.int32),
        pltpu.VMEM((EPT,), jnp.int32),
        pltpu.VMEM((NCHMAX, CH), jnp.int32),
        pltpu.VMEM((NCHMAX, CH), jnp.int32),
        pltpu.VMEM((16,), jnp.int32),
    ],
    compiler_params=pltpu.CompilerParams(needs_layout_passes=False),
)
def _part_kernel(edges_hbm, srcc_hbm, dstc_hbm, cnt_hbm,
                 src_v, dst_v, srcc_v, dstc_v, cnt_v):
    cid = lax.axis_index("c")
    sid = lax.axis_index("s")

    pltpu.sync_copy(edges_hbm.at[pl.ds(sid * EPT, EPT)], src_v)
    pltpu.sync_copy(edges_hbm.at[pl.ds(N_EDGES + sid * EPT, EPT)], dst_v)

    lo = cid * HALF
    lo_vec = jnp.full((16,), 1, jnp.int32) * lo

    def scan(i, off):
        s16 = src_v[pl.ds(i * 16, 16)]
        d16 = dst_v[pl.ds(i * 16, 16)]
        dl = d16 - lo_vec
        m = (dl >= 0) & (dl < HALF)
        mi = m.astype(jnp.int32)
        pos = off + jnp.cumsum(mi) - 1
        plsc.store_scatter(srcc_v, [pos >> 7, pos & 127], s16, mask=m)
        plsc.store_scatter(dstc_v, [pos >> 7, pos & 127], dl, mask=m)
        return off + jnp.sum(mi)

    cnt = lax.fori_loop(0, EPT // 16, scan, jnp.int32(0))

    z16 = jnp.zeros((16,), jnp.int32)
    t16 = jnp.full((16,), 1, jnp.int32) * HALF
    lane = lax.iota(jnp.int32, 16)
    for k in range(CH // 16):
        pos = cnt + 16 * k + lane
        plsc.store_scatter(srcc_v, [pos >> 7, pos & 127], z16)
        plsc.store_scatter(dstc_v, [pos >> 7, pos & 127], t16)

    wid = cid * NS + sid
    pltpu.sync_copy(srcc_v, srcc_hbm.at[wid, pl.ds(0, NCHMAX)])
    pltpu.sync_copy(dstc_v, dstc_hbm.at[wid, pl.ds(0, NCHMAX)])
    cnt_v[...] = jnp.broadcast_to(cnt, (16,))
    pltpu.sync_copy(cnt_v, cnt_hbm.at[pl.ds((cid * NS + sid) * 16, 16)])


@functools.partial(
    pl.kernel,
    out_type=jax.ShapeDtypeStruct((N_NODES, D), jnp.float32),
    mesh=_sc_mesh,
    scratch_types=[
        pltpu.VMEM((NCHMAX, CH), jnp.int32),
        pltpu.VMEM((NCHMAX, CH), jnp.int32),
        pltpu.VMEM((3, CH, D), jnp.float32),
        pltpu.VMEM((16,), jnp.int32),
        pltpu.VMEM_SHARED((ACC_R, D), jnp.float32),
        pltpu.SemaphoreType.DMA,
        pltpu.SemaphoreType.DMA,
    ],
    compiler_params=pltpu.CompilerParams(needs_layout_passes=False),
)
def _agg_kernel(h_hbm, srcc_hbm, dstc_hbm, cnt_hbm, out_hbm,
                src_v, dst_v, rows_v, cnt_v, acc_sh, sem_g, sem_s):
    cid = lax.axis_index("c")
    sid = lax.axis_index("s")

    z16 = jnp.zeros((16,), jnp.float32)

    def fillz(t, _):
        rows_v[0, t // (D // 16), pl.ds((t % (D // 16)) * 16, 16)] = z16
        return 0

    lax.fori_loop(0, CH * (D // 16), fillz, 0)

    pltpu.sync_copy(rows_v.at[0, pl.ds(0, CH)],
                    acc_sh.at[pl.ds(sid * ARPT, CH)])
    pltpu.sync_copy(rows_v.at[0, pl.ds(0, CH)],
                    acc_sh.at[pl.ds(sid * ARPT + CH, CH)])
    pltpu.sync_copy(rows_v.at[0, pl.ds(0, ARPT - 2 * CH)],
                    acc_sh.at[pl.ds(sid * ARPT + 2 * CH, ARPT - 2 * CH)])

    @pl.when(sid == NS - 1)
    def _():
        pltpu.sync_copy(rows_v.at[0, pl.ds(0, ATAIL)],
                        acc_sh.at[pl.ds(HALF - ATAIL, ATAIL)])

    plsc.subcore_barrier()

    wid = cid * NS + sid
    pltpu.sync_copy(cnt_hbm.at[pl.ds(wid * 16, 16)], cnt_v)
    cnt = jnp.max(cnt_v[...])
    nch = (cnt + CH - 1) // CH

    pltpu.sync_copy(srcc_hbm.at[wid, pl.ds(0, NCHMAX)], src_v)
    pltpu.sync_copy(dstc_hbm.at[wid, pl.ds(0, NCHMAX)], dst_v)

    def _gather(f):
        pltpu.async_copy(h_hbm.at[src_v.at[f, pl.ds(0, CH)]],
                         rows_v.at[f % 3, pl.ds(0, CH)], sem_g)

    def _wait_g(f):
        pltpu.make_async_copy(h_hbm.at[src_v.at[f, pl.ds(0, CH)]],
                              rows_v.at[f % 3, pl.ds(0, CH)], sem_g).wait()

    def _scatter(f):
        pltpu.async_copy(rows_v.at[f % 3, pl.ds(0, CH)],
                         acc_sh.at[dst_v.at[f, pl.ds(0, CH)]], sem_s, add=True)

    def _wait_s():
        pltpu.make_async_copy(rows_v.at[0, pl.ds(0, CH)],
                              acc_sh.at[dst_v.at[0, pl.ds(0, CH)]], sem_s).wait()

    for k in range(3):
        @pl.when(k < nch)
        def _():
            _gather(jnp.int32(k))

    def chunk(j, _):
        _wait_g(j)
        _scatter(j)

        @pl.when(j >= 1)
        def _():
            _wait_s()

            @pl.when(j + 2 < nch)
            def _():
                _gather(j + 2)
        return 0

    lax.fori_loop(0, nch, chunk, 0)

    @pl.when(nch >= 1)
    def _():
        _wait_s()
    plsc.subcore_barrier()

    pltpu.sync_copy(acc_sh.at[pl.ds(sid * ARPT, ARPT)],
                    out_hbm.at[pl.ds(cid * HALF + sid * ARPT, ARPT)])

    @pl.when(sid == NS - 1)
    def _():
        pltpu.sync_copy(acc_sh.at[pl.ds(HALF - ATAIL, ATAIL)],
                        out_hbm.at[pl.ds(cid * HALF + HALF - ATAIL, ATAIL)])


@functools.partial(
    pl.kernel,
    out_type=(
        jax.ShapeDtypeStruct((NC, HR, 128), jnp.float32),
        jax.ShapeDtypeStruct((NC * NS, NCHMAX, CH), jnp.int32),
        jax.ShapeDtypeStruct((NC * NS, NCHMAX, CH), jnp.int32),
        jax.ShapeDtypeStruct((NC * NS * 16,), jnp.int32),
    ),
    mesh=_sc_mesh,
    scratch_types=[
        pltpu.VMEM((EPT,), jnp.int32),
        pltpu.VMEM((EPT,), jnp.int32),
        pltpu.VMEM((NCHMAX, CH), jnp.int32),
        pltpu.VMEM((NCHMAX, CH), jnp.int32),
        pltpu.VMEM((16,), jnp.int32),
        pltpu.VMEM((HR, 128), jnp.float32),
        pltpu.VMEM((HPT, 128), jnp.float32),
        pltpu.VMEM((HPT, 128), jnp.float32),
        pltpu.VMEM_SHARED((NS, HR, 128), jnp.float32),
    ],
    compiler_params=pltpu.CompilerParams(needs_layout_passes=False),
)
def _prep_kernel(edges_hbm, deg_hbm, srcc_hbm, dstc_hbm, cnt_hbm,
                 src_v, dst_v, srcc_v, dstc_v, cnt_v,
                 hist_v, hacc_v, hbuf_v, stage_sh):
    cid = lax.axis_index("c")
    sid = lax.axis_index("s")

    one16 = jnp.ones((16,), jnp.float32)
    z16f = jnp.zeros((16,), jnp.float32)

    pltpu.sync_copy(edges_hbm.at[pl.ds(sid * EPT, EPT)], src_v)
    pltpu.sync_copy(edges_hbm.at[pl.ds(N_EDGES + sid * EPT, EPT)], dst_v)

    def fillz(i, _):
        hist_v[i // 8, pl.ds((i % 8) * 16, 16)] = z16f
        return 0

    lax.fori_loop(0, HR * 8, fillz, 0)

    lo = cid * HALF
    lo_vec = jnp.full((16,), 1, jnp.int32) * lo

    def scan(i, off):
        s16 = src_v[pl.ds(i * 16, 16)]
        d16 = dst_v[pl.ds(i * 16, 16)]
        h16 = jnp.where(cid == 0, s16, d16)
        plsc.addupdate_scatter(hist_v, [h16 >> 7, h16 & 127], one16)
        dl = d16 - lo_vec
        m = (dl >= 0) & (dl < HALF)
        mi = m.astype(jnp.int32)
        pos = off + jnp.cumsum(mi) - 1
        plsc.store_scatter(srcc_v, [pos >> 7, pos & 127], s16, mask=m)
        plsc.store_scatter(dstc_v, [pos >> 7, pos & 127], dl, mask=m)
        return off + jnp.sum(mi)

    cnt = lax.fori_loop(0, EPT // 16, scan, jnp.int32(0))

    z16 = jnp.zeros((16,), jnp.int32)
    t16 = jnp.full((16,), 1, jnp.int32) * HALF
    lane = lax.iota(jnp.int32, 16)
    for k in range(CH // 16):
        pos = cnt + 16 * k + lane
        plsc.store_scatter(srcc_v, [pos >> 7, pos & 127], z16)
        plsc.store_scatter(dstc_v, [pos >> 7, pos & 127], t16)

    wid = cid * NS + sid
    pltpu.sync_copy(srcc_v, srcc_hbm.at[wid, pl.ds(0, NCHMAX)])
    pltpu.sync_copy(dstc_v, dstc_hbm.at[wid, pl.ds(0, NCHMAX)])
    cnt_v[...] = jnp.broadcast_to(cnt, (16,))
    pltpu.sync_copy(cnt_v, cnt_hbm.at[pl.ds(wid * 16, 16)])

    pltpu.sync_copy(hist_v, stage_sh.at[sid])
    plsc.subcore_barrier()

    def fillza(i, _):
        hacc_v[i // 8, pl.ds((i % 8) * 16, 16)] = z16f
        return 0

    lax.fori_loop(0, HPT * 8, fillza, 0)

    for s in range(NS):
        pltpu.sync_copy(stage_sh.at[s, pl.ds(sid * HPT, HPT)], hbuf_v)

        def addp(i, _):
            r, c = i // 8, (i % 8) * 16
            hacc_v[r, pl.ds(c, 16)] = hacc_v[r, pl.ds(c, 16)] + hbuf_v[r, pl.ds(c, 16)]
            return 0

        lax.fori_loop(0, HPT * 8, addp, 0)

    pltpu.sync_copy(hacc_v, deg_hbm.at[cid, pl.ds(sid * HPT, HPT)])


_BLK = 1000


def _norm(deg_blk):
    return lax.rsqrt(jnp.maximum(deg_blk[...], 1.0))


def _pre1_body(x_ref, dgo_ref, o_ref):
    o_ref[...] = x_ref[...] * _norm(dgo_ref)


def _post1_body(p_ref, dgi_ref, dgo_ref, w_ref, b_ref, o_ref):
    a = p_ref[...] * _norm(dgi_ref)
    z = jnp.dot(a, w_ref[...], preferred_element_type=jnp.float32) + b_ref[...]
    o_ref[...] = jnp.maximum(z, 0.0) * _norm(dgo_ref)


def _post2_body(p_ref, dgi_ref, w_ref, b_ref, o_ref):
    a = p_ref[...] * _norm(dgi_ref)
    o_ref[...] = jnp.dot(a, w_ref[...], preferred_element_type=jnp.float32) + b_ref[...]


def _row_blocked(width):
    return pl.BlockSpec((_BLK, width), lambda i: (i, 0))


_SPEC_W = pl.BlockSpec((D, D), lambda i: (0, 0))
_SPEC_B = pl.BlockSpec((1, D), lambda i: (0, 0))
_OUT_SHAPE = jax.ShapeDtypeStruct((N_NODES, D), jnp.float32)
_GRID = (N_NODES // _BLK,)


def kernel(x, edge_index, W1, b1, W2, b2):
    eflat = edge_index.reshape(2 * N_EDGES)
    deg, srcc, dstc, cnts = _prep_kernel(eflat)
    degp = deg.reshape(NC, HR * 128)[:, :N_NODES]
    dgo = degp[0][:, None]
    dgi = degp[1][:, None]

    h1 = pl.pallas_call(
        _pre1_body,
        grid=_GRID,
        in_specs=[_row_blocked(D), _row_blocked(1)],
        out_specs=_row_blocked(D),
        out_shape=_OUT_SHAPE,
    )(x, dgo)

    p1 = _agg_kernel(h1, srcc, dstc, cnts)

    h2 = pl.pallas_call(
        _post1_body,
        grid=_GRID,
        in_specs=[_row_blocked(D), _row_blocked(1), _row_blocked(1),
                  _SPEC_W, _SPEC_B],
        out_specs=_row_blocked(D),
        out_shape=_OUT_SHAPE,
    )(p1, dgi, dgo, W1, b1.reshape(1, D))

    p2 = _agg_kernel(h2, srcc, dstc, cnts)

    out = pl.pallas_call(
        _post2_body,
        grid=_GRID,
        in_specs=[_row_blocked(D), _row_blocked(1), _SPEC_W, _SPEC_B],
        out_specs=_row_blocked(D),
        out_shape=_OUT_SHAPE,
    )(p2, dgi, W2, b2.reshape(1, D))

    return out
